# Initial kernel scaffold; baseline (speedup 1.0000x reference)
import jax
import jax.numpy as jnp
from jax.experimental import pallas as pl


def kernel(x, edge_index, batch, W_proj, b_proj, W_g0, b_g0, W_g1, b_g1, W_g2, b_g2, W_c1, b_c1, W_c2, b_c2):
    raise NotImplementedError("write your pallas kernel here")


# speedup vs baseline: 11.3240x; 11.3240x over previous
import functools

import jax
import jax.numpy as jnp
from jax import lax
from jax.experimental import pallas as pl
from jax.experimental.pallas import tpu as pltpu
from jax.experimental.pallas import tpu_sc as plsc

N = 10000
E = 320000
D = 128
NB = 16
NC = 2
NS = 16
NW = NC * NS
EW = E // NW
K = 80
NCH = EW // K
RB = 10
RBS = 1024
NP = 10240
RPT = NP // NS

_F32 = jnp.float32


def _sc_mesh():
    return plsc.VectorSubcoreMesh(core_axis_name="c", subcore_axis_name="s")


def _hist_call(dst):
    @functools.partial(
        pl.kernel,
        out_type=jax.ShapeDtypeStruct((NC, NP), _F32),
        mesh=_sc_mesh(),
        scratch_types=[
            pltpu.VMEM_SHARED((NP,), _F32),
            pltpu.VMEM((640,), _F32),
            pltpu.VMEM((K,), _F32),
            pltpu.VMEM((K,), jnp.int32),
        ],
    )
    def hist_kernel(dst_hbm, out_hbm, hist_sh, zbuf, ones_v, didx):
        c = lax.axis_index("c")
        s = lax.axis_index("s")

        def fill(i, _):
            zbuf[pl.ds(i * 16, 16)] = jnp.zeros((16,), _F32)
            return 0

        lax.fori_loop(0, 40, fill, 0)
        for i in range(K // 16):
            ones_v[pl.ds(i * 16, 16)] = jnp.ones((16,), _F32)

        pltpu.sync_copy(zbuf, hist_sh.at[pl.ds(s * 640, 640)])
        plsc.subcore_barrier()

        base = (c * NS + s) * EW

        def body(it, _):
            pltpu.sync_copy(dst_hbm.at[pl.ds(base + it * K, K)], didx)
            pltpu.sync_copy(ones_v, hist_sh.at[didx], add=True)
            return 0

        lax.fori_loop(0, NCH, body, 0)
        plsc.subcore_barrier()
        pltpu.sync_copy(hist_sh.at[pl.ds(s * 640, 640)],
                        out_hbm.at[c, pl.ds(s * 640, 640)])

    return hist_kernel(dst)


def _agg_call(hp, src, dst):
    @functools.partial(
        pl.kernel,
        out_type=jax.ShapeDtypeStruct((NC, NP, D), _F32),
        mesh=_sc_mesh(),
        scratch_types=[
            pltpu.VMEM_SHARED((NP, D), _F32),
            pltpu.VMEM((128, D), _F32),
            pltpu.VMEM((K,), jnp.int32),
            pltpu.VMEM((K,), jnp.int32),
            pltpu.VMEM((K, D), _F32),
            pltpu.SemaphoreType.DMA,
        ],
    )
    def agg_kernel(hp_hbm, src_hbm, dst_hbm, out_hbm, acc, zbuf, sidx, didx,
                   rows, sem):
        c = lax.axis_index("c")
        s = lax.axis_index("s")

        def fill(i, _):
            zbuf[i // 8, pl.ds((i % 8) * 16, 16)] = jnp.zeros((16,), _F32)
            return 0

        lax.fori_loop(0, 1024, fill, 0)
        for j in range(5):
            pltpu.sync_copy(zbuf, acc.at[pl.ds(s * RPT + j * 128, 128)])
        plsc.subcore_barrier()

        base = (c * NS + s) * EW

        def body(it, _):
            off = base + it * K
            pltpu.sync_copy(src_hbm.at[pl.ds(off, K)], sidx)
            pltpu.sync_copy(dst_hbm.at[pl.ds(off, K)], didx)
            pltpu.async_copy(hp_hbm.at[sidx], rows, sem).wait()
            pltpu.async_copy(rows, acc.at[didx], sem, add=True).wait()
            return 0

        lax.fori_loop(0, NCH, body, 0)
        plsc.subcore_barrier()
        pltpu.sync_copy(acc.at[pl.ds(s * RPT, RPT)],
                        out_hbm.at[c, pl.ds(s * RPT, RPT)])

    return agg_kernel(hp, src, dst)


def _tc0_call(x, hist, W_proj, b_proj, W_g0):
    def body(x_ref, hist_ref, wp_ref, bp_ref, wg_ref, hp_ref, dinv_ref):
        dinv = lax.rsqrt(1.0 + hist_ref[0] + hist_ref[1])
        h = jnp.maximum(
            jnp.dot(x_ref[...], wp_ref[...], preferred_element_type=_F32)
            + bp_ref[...], 0.0)
        hp_ref[...] = jnp.dot(dinv * h, wg_ref[...],
                              preferred_element_type=_F32)
        dinv_ref[...] = dinv

    return pl.pallas_call(
        body,
        grid=(RB,),
        in_specs=[
            pl.BlockSpec((RBS, D), lambda i: (i, 0)),
            pl.BlockSpec((NC, RBS, 1), lambda i: (0, i, 0)),
            pl.BlockSpec((D, D), lambda i: (0, 0)),
            pl.BlockSpec((1, D), lambda i: (0, 0)),
            pl.BlockSpec((D, D), lambda i: (0, 0)),
        ],
        out_specs=[
            pl.BlockSpec((RBS, D), lambda i: (i, 0)),
            pl.BlockSpec((RBS, 1), lambda i: (i, 0)),
        ],
        out_shape=[
            jax.ShapeDtypeStruct((N, D), _F32),
            jax.ShapeDtypeStruct((N, 1), _F32),
        ],
    )(x, hist, W_proj, b_proj, W_g0)


def _tcmid_call(s2, hp, dinv, b_prev, W_next):
    def body(s_ref, hp_ref, dinv_ref, b_ref, w_ref, out_ref):
        dinv = dinv_ref[...]
        h = jnp.maximum(
            dinv * (s_ref[0] + s_ref[1] + hp_ref[...]) + b_ref[...], 0.0)
        out_ref[...] = jnp.dot(dinv * h, w_ref[...],
                               preferred_element_type=_F32)

    return pl.pallas_call(
        body,
        grid=(RB,),
        in_specs=[
            pl.BlockSpec((NC, RBS, D), lambda i: (0, i, 0)),
            pl.BlockSpec((RBS, D), lambda i: (i, 0)),
            pl.BlockSpec((RBS, 1), lambda i: (i, 0)),
            pl.BlockSpec((1, D), lambda i: (0, 0)),
            pl.BlockSpec((D, D), lambda i: (0, 0)),
        ],
        out_specs=pl.BlockSpec((RBS, D), lambda i: (i, 0)),
        out_shape=jax.ShapeDtypeStruct((N, D), _F32),
    )(s2, hp, dinv, b_prev, W_next)


def _tcf_call(s2, hp, dinv, b_prev, batch, W_c1, b_c1, W_c2, b_c2):
    def body(s_ref, hp_ref, dinv_ref, b_ref, batch_ref, wc1_ref, bc1_ref,
             wc2_ref, bc2_ref, out_ref, acc, cnt):
        i = pl.program_id(0)

        @pl.when(i == 0)
        def _():
            acc[...] = jnp.zeros((NB, D), _F32)
            cnt[...] = jnp.zeros((NB, 1), _F32)

        dinv = dinv_ref[...]
        h = jnp.maximum(
            dinv * (s_ref[0] + s_ref[1] + hp_ref[...]) + b_ref[...], 0.0)
        valid_col = (i * RBS
                     + lax.broadcasted_iota(jnp.int32, (RBS, 1), 0)) < N
        h = jnp.where(valid_col, h, 0.0)
        valid_row = (i * RBS
                     + lax.broadcasted_iota(jnp.int32, (1, RBS), 1)) < N
        bb = batch_ref[pl.ds(i * RBS, RBS)]
        mask = jnp.where(
            (lax.broadcasted_iota(jnp.int32, (NB, RBS), 0) == bb[None, :])
            & valid_row, 1.0, 0.0)
        acc[...] += jnp.dot(mask, h, preferred_element_type=_F32)
        cnt[...] += jnp.sum(mask, axis=1, keepdims=True)

        @pl.when(i == RB - 1)
        def _():
            pooled = acc[...] / jnp.maximum(cnt[...], 1.0)
            z = jnp.maximum(
                jnp.dot(pooled, wc1_ref[...], preferred_element_type=_F32)
                + bc1_ref[...], 0.0)
            out_ref[...] = jnp.dot(z, wc2_ref[...],
                                   preferred_element_type=_F32) + bc2_ref[...]

    return pl.pallas_call(
        body,
        grid=(RB,),
        in_specs=[
            pl.BlockSpec((NC, RBS, D), lambda i: (0, i, 0)),
            pl.BlockSpec((RBS, D), lambda i: (i, 0)),
            pl.BlockSpec((RBS, 1), lambda i: (i, 0)),
            pl.BlockSpec((1, D), lambda i: (0, 0)),
            pl.BlockSpec((NP,), lambda i: (0,)),
            pl.BlockSpec((D, D // 2), lambda i: (0, 0)),
            pl.BlockSpec((1, D // 2), lambda i: (0, 0)),
            pl.BlockSpec((D // 2, 2), lambda i: (0, 0)),
            pl.BlockSpec((1, 2), lambda i: (0, 0)),
        ],
        out_specs=pl.BlockSpec((NB, 2), lambda i: (0, 0)),
        out_shape=jax.ShapeDtypeStruct((NB, 2), _F32),
        scratch_shapes=[
            pltpu.VMEM((NB, D), _F32),
            pltpu.VMEM((NB, 1), _F32),
        ],
    )(s2, hp, dinv, b_prev, batch, W_c1, b_c1, W_c2, b_c2)


def kernel(x, edge_index, batch, W_proj, b_proj, W_g0, b_g0, W_g1, b_g1,
           W_g2, b_g2, W_c1, b_c1, W_c2, b_c2):
    src = edge_index[0]
    dst = edge_index[1]
    hist = _hist_call(dst).reshape(NC, NP, 1)
    batch_p = jnp.pad(batch, (0, NP - N))
    hp0, dinv = _tc0_call(x, hist, W_proj, b_proj.reshape(1, D), W_g0)
    s0 = _agg_call(hp0, src, dst)
    hp1 = _tcmid_call(s0, hp0, dinv, b_g0.reshape(1, D), W_g1)
    s1 = _agg_call(hp1, src, dst)
    hp2 = _tcmid_call(s1, hp1, dinv, b_g1.reshape(1, D), W_g2)
    s2 = _agg_call(hp2, src, dst)
    return _tcf_call(s2, hp2, dinv, b_g2.reshape(1, D), batch_p,
                     W_c1, b_c1.reshape(1, D // 2), W_c2, b_c2.reshape(1, 2))


# speedup vs baseline: 26.8001x; 2.3667x over previous
import functools

import jax
import jax.numpy as jnp
from jax import lax
from jax.experimental import pallas as pl
from jax.experimental.pallas import tpu as pltpu
from jax.experimental.pallas import tpu_sc as plsc

N = 10000
E = 320000
D = 128
NB = 16
NC = 2
NS = 16
NW = NC * NS
EW = E // NW
K = 80
NCH = EW // K
RB = 10
RBS = 1024
NP = 10240
RPT = NP // NS

_F32 = jnp.float32


def _sc_mesh():
    return plsc.VectorSubcoreMesh(core_axis_name="c", subcore_axis_name="s")


def _hist_call(dst3):
    @functools.partial(
        pl.kernel,
        out_type=jax.ShapeDtypeStruct((NC, NP), _F32),
        mesh=_sc_mesh(),
        scratch_types=[
            pltpu.VMEM_SHARED((NP,), _F32),
            pltpu.VMEM((640,), _F32),
            pltpu.VMEM((K,), _F32),
            pltpu.VMEM((NCH, K), jnp.int32),
            pltpu.SemaphoreType.DMA,
        ],
    )
    def hist_kernel(dst_hbm, out_hbm, hist_sh, zbuf, ones_v, didx, sem):
        c = lax.axis_index("c")
        s = lax.axis_index("s")
        w = c * NS + s

        def fill(i, _):
            zbuf[pl.ds(i * 16, 16)] = jnp.zeros((16,), _F32)
            return 0

        lax.fori_loop(0, 40, fill, 0)
        for i in range(K // 16):
            ones_v[pl.ds(i * 16, 16)] = jnp.ones((16,), _F32)

        pltpu.sync_copy(zbuf, hist_sh.at[pl.ds(s * 640, 640)])
        pltpu.sync_copy(dst_hbm.at[w], didx)
        plsc.subcore_barrier()

        def fire(it, _):
            pltpu.async_copy(ones_v, hist_sh.at[didx.at[it]], sem, add=True)
            return 0

        lax.fori_loop(0, NCH, fire, 0)

        def drain(it, _):
            pltpu.make_async_copy(ones_v, hist_sh.at[didx.at[it]], sem).wait()
            return 0

        lax.fori_loop(0, NCH, drain, 0)
        plsc.subcore_barrier()
        pltpu.sync_copy(hist_sh.at[pl.ds(s * 640, 640)],
                        out_hbm.at[c, pl.ds(s * 640, 640)])

    return hist_kernel(dst3)


NBUF = 2
NFULL = NCH // NBUF
NREM = NCH % NBUF


def _agg_call(hp, src3, dst3):
    @functools.partial(
        pl.kernel,
        out_type=jax.ShapeDtypeStruct((NC, NP, D), _F32),
        mesh=_sc_mesh(),
        scratch_types=[
            pltpu.VMEM_SHARED((NP, D), _F32),
            pltpu.VMEM((8, D), _F32),
            pltpu.VMEM((EW,), jnp.int32),
            pltpu.VMEM((NCH, K), jnp.int32),
        ] + [pltpu.VMEM((K, D), _F32)] * NBUF
          + [pltpu.SemaphoreType.DMA] * (2 * NBUF),
    )
    def agg_kernel(hp_hbm, src_hbm, dst_hbm, out_hbm, acc, zbuf, sidx, didx,
                   r0, r1, g0, g1, t0, t1):
        rows = [r0, r1]
        gsem = [g0, g1]
        ssem = [t0, t1]
        c = lax.axis_index("c")
        s = lax.axis_index("s")
        w = c * NS + s

        def fill(i, _):
            zbuf[i // 8, pl.ds((i % 8) * 16, 16)] = jnp.zeros((16,), _F32)
            return 0

        lax.fori_loop(0, 64, fill, 0)

        def zero(j, _):
            pltpu.sync_copy(zbuf, acc.at[pl.ds(s * RPT + j * 8, 8)])
            return 0

        lax.fori_loop(0, RPT // 8, zero, 0)
        pltpu.sync_copy(src_hbm.at[pl.ds(w * EW, EW)], sidx)
        pltpu.sync_copy(dst_hbm.at[w], didx)
        plsc.subcore_barrier()

        def gidx(it):
            return sidx.at[pl.ds(it * K, K)]

        for b in range(NBUF):
            pltpu.async_copy(hp_hbm.at[gidx(b)], rows[b], gsem[b])

        def step(it, b, prefetch):
            pltpu.make_async_copy(hp_hbm.at[gidx(it)], rows[b],
                                  gsem[b]).wait()
            pltpu.async_copy(rows[b], acc.at[didx.at[it]], ssem[b], add=True)
            pltpu.make_async_copy(rows[b], acc.at[didx.at[it]],
                                  ssem[b]).wait()
            if prefetch:
                pltpu.async_copy(hp_hbm.at[gidx(it + NBUF)], rows[b],
                                 gsem[b])

        def outer(g, _):
            for b in range(NBUF):
                step(g * NBUF + b, b, True)
            return 0

        lax.fori_loop(0, NFULL - 1, outer, 0)
        for b in range(NBUF):
            step((NFULL - 1) * NBUF + b, b, b < NREM)
        for b in range(NREM):
            step(NFULL * NBUF + b, b, False)

        plsc.subcore_barrier()
        pltpu.sync_copy(acc.at[pl.ds(s * RPT, RPT)],
                        out_hbm.at[c, pl.ds(s * RPT, RPT)])

    return agg_kernel(hp, src3, dst3)


def _tc0_call(x, hist, W_proj, b_proj, W_g0):
    def body(x_ref, hist_ref, wp_ref, bp_ref, wg_ref, hp_ref, dinv_ref):
        dinv = lax.rsqrt(1.0 + hist_ref[0] + hist_ref[1])
        h = jnp.maximum(
            jnp.dot(x_ref[...], wp_ref[...], preferred_element_type=_F32)
            + bp_ref[...], 0.0)
        hp_ref[...] = jnp.dot(dinv * h, wg_ref[...],
                              preferred_element_type=_F32)
        dinv_ref[...] = dinv

    return pl.pallas_call(
        body,
        grid=(RB,),
        in_specs=[
            pl.BlockSpec((RBS, D), lambda i: (i, 0)),
            pl.BlockSpec((NC, RBS, 1), lambda i: (0, i, 0)),
            pl.BlockSpec((D, D), lambda i: (0, 0)),
            pl.BlockSpec((1, D), lambda i: (0, 0)),
            pl.BlockSpec((D, D), lambda i: (0, 0)),
        ],
        out_specs=[
            pl.BlockSpec((RBS, D), lambda i: (i, 0)),
            pl.BlockSpec((RBS, 1), lambda i: (i, 0)),
        ],
        out_shape=[
            jax.ShapeDtypeStruct((N, D), _F32),
            jax.ShapeDtypeStruct((N, 1), _F32),
        ],
    )(x, hist, W_proj, b_proj, W_g0)


def _tcmid_call(s2, hp, dinv, b_prev, W_next):
    def body(s_ref, hp_ref, dinv_ref, b_ref, w_ref, out_ref):
        dinv = dinv_ref[...]
        h = jnp.maximum(
            dinv * (s_ref[0] + s_ref[1] + hp_ref[...]) + b_ref[...], 0.0)
        out_ref[...] = jnp.dot(dinv * h, w_ref[...],
                               preferred_element_type=_F32)

    return pl.pallas_call(
        body,
        grid=(RB,),
        in_specs=[
            pl.BlockSpec((NC, RBS, D), lambda i: (0, i, 0)),
            pl.BlockSpec((RBS, D), lambda i: (i, 0)),
            pl.BlockSpec((RBS, 1), lambda i: (i, 0)),
            pl.BlockSpec((1, D), lambda i: (0, 0)),
            pl.BlockSpec((D, D), lambda i: (0, 0)),
        ],
        out_specs=pl.BlockSpec((RBS, D), lambda i: (i, 0)),
        out_shape=jax.ShapeDtypeStruct((N, D), _F32),
    )(s2, hp, dinv, b_prev, W_next)


def _tcf_call(s2, hp, dinv, b_prev, batch, W_c1, b_c1, W_c2, b_c2):
    def body(s_ref, hp_ref, dinv_ref, b_ref, batch_ref, wc1_ref, bc1_ref,
             wc2_ref, bc2_ref, out_ref, acc, cnt):
        i = pl.program_id(0)

        @pl.when(i == 0)
        def _():
            acc[...] = jnp.zeros((NB, D), _F32)
            cnt[...] = jnp.zeros((NB, 1), _F32)

        dinv = dinv_ref[...]
        h = jnp.maximum(
            dinv * (s_ref[0] + s_ref[1] + hp_ref[...]) + b_ref[...], 0.0)
        valid_col = (i * RBS
                     + lax.broadcasted_iota(jnp.int32, (RBS, 1), 0)) < N
        h = jnp.where(valid_col, h, 0.0)
        valid_row = (i * RBS
                     + lax.broadcasted_iota(jnp.int32, (1, RBS), 1)) < N
        bb = batch_ref[pl.ds(i * RBS, RBS)]
        mask = jnp.where(
            (lax.broadcasted_iota(jnp.int32, (NB, RBS), 0) == bb[None, :])
            & valid_row, 1.0, 0.0)
        acc[...] += jnp.dot(mask, h, preferred_element_type=_F32)
        cnt[...] += jnp.sum(mask, axis=1, keepdims=True)

        @pl.when(i == RB - 1)
        def _():
            pooled = acc[...] / jnp.maximum(cnt[...], 1.0)
            z = jnp.maximum(
                jnp.dot(pooled, wc1_ref[...], preferred_element_type=_F32)
                + bc1_ref[...], 0.0)
            out_ref[...] = jnp.dot(z, wc2_ref[...],
                                   preferred_element_type=_F32) + bc2_ref[...]

    return pl.pallas_call(
        body,
        grid=(RB,),
        in_specs=[
            pl.BlockSpec((NC, RBS, D), lambda i: (0, i, 0)),
            pl.BlockSpec((RBS, D), lambda i: (i, 0)),
            pl.BlockSpec((RBS, 1), lambda i: (i, 0)),
            pl.BlockSpec((1, D), lambda i: (0, 0)),
            pl.BlockSpec((NP,), lambda i: (0,)),
            pl.BlockSpec((D, D // 2), lambda i: (0, 0)),
            pl.BlockSpec((1, D // 2), lambda i: (0, 0)),
            pl.BlockSpec((D // 2, 2), lambda i: (0, 0)),
            pl.BlockSpec((1, 2), lambda i: (0, 0)),
        ],
        out_specs=pl.BlockSpec((NB, 2), lambda i: (0, 0)),
        out_shape=jax.ShapeDtypeStruct((NB, 2), _F32),
        scratch_shapes=[
            pltpu.VMEM((NB, D), _F32),
            pltpu.VMEM((NB, 1), _F32),
        ],
    )(s2, hp, dinv, b_prev, batch, W_c1, b_c1, W_c2, b_c2)


def kernel(x, edge_index, batch, W_proj, b_proj, W_g0, b_g0, W_g1, b_g1,
           W_g2, b_g2, W_c1, b_c1, W_c2, b_c2):
    src = edge_index[0]
    dst3 = edge_index[1].reshape(NW, NCH, K)
    hist = _hist_call(dst3).reshape(NC, NP, 1)
    batch_p = jnp.pad(batch, (0, NP - N))
    hp0, dinv = _tc0_call(x, hist, W_proj, b_proj.reshape(1, D), W_g0)
    s0 = _agg_call(hp0, src, dst3)
    hp1 = _tcmid_call(s0, hp0, dinv, b_g0.reshape(1, D), W_g1)
    s1 = _agg_call(hp1, src, dst3)
    hp2 = _tcmid_call(s1, hp1, dinv, b_g1.reshape(1, D), W_g2)
    s2 = _agg_call(hp2, src, dst3)
    return _tcf_call(s2, hp2, dinv, b_g2.reshape(1, D), batch_p,
                     W_c1, b_c1.reshape(1, D // 2), W_c2, b_c2.reshape(1, 2))


# speedup vs baseline: 26.9246x; 1.0046x over previous
import functools

import jax
import jax.numpy as jnp
from jax import lax
from jax.experimental import pallas as pl
from jax.experimental.pallas import tpu as pltpu
from jax.experimental.pallas import tpu_sc as plsc

N = 10000
E = 320000
D = 128
NB = 16
NC = 2
NS = 16
NW = NC * NS
EW = E // NW
K = 80
NCH = EW // K
RB = 10
RBS = 1024
NP = 10240
RPT = NP // NS

_F32 = jnp.float32


def _sc_mesh():
    return plsc.VectorSubcoreMesh(core_axis_name="c", subcore_axis_name="s")


def _hist_call(dst3):
    @functools.partial(
        pl.kernel,
        out_type=jax.ShapeDtypeStruct((NC, NP), _F32),
        mesh=_sc_mesh(),
        scratch_types=[
            pltpu.VMEM_SHARED((NP,), _F32),
            pltpu.VMEM((640,), _F32),
            pltpu.VMEM((K,), _F32),
            pltpu.VMEM((NCH, K), jnp.int32),
            pltpu.SemaphoreType.DMA,
        ],
    )
    def hist_kernel(dst_hbm, out_hbm, hist_sh, zbuf, ones_v, didx, sem):
        c = lax.axis_index("c")
        s = lax.axis_index("s")
        w = c * NS + s

        def fill(i, _):
            zbuf[pl.ds(i * 16, 16)] = jnp.zeros((16,), _F32)
            return 0

        lax.fori_loop(0, 40, fill, 0)
        for i in range(K // 16):
            ones_v[pl.ds(i * 16, 16)] = jnp.ones((16,), _F32)

        pltpu.sync_copy(zbuf, hist_sh.at[pl.ds(s * 640, 640)])
        pltpu.sync_copy(dst_hbm.at[w], didx)
        plsc.subcore_barrier()

        def fire(it, _):
            pltpu.async_copy(ones_v, hist_sh.at[didx.at[it]], sem, add=True)
            return 0

        lax.fori_loop(0, NCH, fire, 0)

        def drain(it, _):
            pltpu.make_async_copy(ones_v, hist_sh.at[didx.at[it]], sem).wait()
            return 0

        lax.fori_loop(0, NCH, drain, 0)
        plsc.subcore_barrier()
        pltpu.sync_copy(hist_sh.at[pl.ds(s * 640, 640)],
                        out_hbm.at[c, pl.ds(s * 640, 640)])

    return hist_kernel(dst3)


NBUF = 2
NFULL = NCH // NBUF
NREM = NCH % NBUF


def _agg_call(hp, src3, dst3):
    @functools.partial(
        pl.kernel,
        out_type=jax.ShapeDtypeStruct((NC, NP, D), _F32),
        mesh=_sc_mesh(),
        scratch_types=[
            pltpu.VMEM_SHARED((NP, D), _F32),
            pltpu.VMEM((8, D), _F32),
            pltpu.VMEM((EW,), jnp.int32),
            pltpu.VMEM((NCH, K), jnp.int32),
        ] + [pltpu.VMEM((K, D), _F32)] * NBUF
          + [pltpu.SemaphoreType.DMA] * (2 * NBUF),
    )
    def agg_kernel(hp_hbm, src_hbm, dst_hbm, out_hbm, acc, zbuf, sidx, didx,
                   r0, r1, g0, g1, t0, t1):
        rows = [r0, r1]
        gsem = [g0, g1]
        ssem = [t0, t1]
        c = lax.axis_index("c")
        s = lax.axis_index("s")
        w = c * NS + s

        def gidx(it):
            return sidx.at[pl.ds(it * K, K)]

        pltpu.sync_copy(src_hbm.at[pl.ds(w * EW, EW)], sidx)
        for b in range(NBUF):
            pltpu.async_copy(hp_hbm.at[gidx(b)], rows[b], gsem[b])
        pltpu.sync_copy(dst_hbm.at[w], didx)

        def fill(i, _):
            zbuf[i // 8, pl.ds((i % 8) * 16, 16)] = jnp.zeros((16,), _F32)
            return 0

        lax.fori_loop(0, 64, fill, 0)

        def zero(j, _):
            pltpu.sync_copy(zbuf, acc.at[pl.ds(s * RPT + j * 8, 8)])
            return 0

        lax.fori_loop(0, RPT // 8, zero, 0)
        plsc.subcore_barrier()

        def step(it, b, prefetch):
            pltpu.make_async_copy(hp_hbm.at[gidx(it)], rows[b],
                                  gsem[b]).wait()
            pltpu.async_copy(rows[b], acc.at[didx.at[it]], ssem[b], add=True)
            pltpu.make_async_copy(rows[b], acc.at[didx.at[it]],
                                  ssem[b]).wait()
            if prefetch:
                pltpu.async_copy(hp_hbm.at[gidx(it + NBUF)], rows[b],
                                 gsem[b])

        def outer(g, _):
            for b in range(NBUF):
                step(g * NBUF + b, b, True)
            return 0

        lax.fori_loop(0, NFULL - 1, outer, 0)
        for b in range(NBUF):
            step((NFULL - 1) * NBUF + b, b, b < NREM)
        for b in range(NREM):
            step(NFULL * NBUF + b, b, False)

        plsc.subcore_barrier()
        pltpu.sync_copy(acc.at[pl.ds(s * RPT, RPT)],
                        out_hbm.at[c, pl.ds(s * RPT, RPT)])

    return agg_kernel(hp, src3, dst3)


def _tc0_call(x, hist, W_proj, b_proj, W_g0):
    def body(x_ref, hist_ref, wp_ref, bp_ref, wg_ref, hp_ref, dinv_ref):
        dinv = lax.rsqrt(1.0 + hist_ref[0] + hist_ref[1])
        h = jnp.maximum(
            jnp.dot(x_ref[...], wp_ref[...], preferred_element_type=_F32)
            + bp_ref[...], 0.0)
        hp_ref[...] = jnp.dot(dinv * h, wg_ref[...],
                              preferred_element_type=_F32)
        dinv_ref[...] = dinv

    return pl.pallas_call(
        body,
        grid=(RB,),
        in_specs=[
            pl.BlockSpec((RBS, D), lambda i: (i, 0)),
            pl.BlockSpec((NC, RBS, 1), lambda i: (0, i, 0)),
            pl.BlockSpec((D, D), lambda i: (0, 0)),
            pl.BlockSpec((1, D), lambda i: (0, 0)),
            pl.BlockSpec((D, D), lambda i: (0, 0)),
        ],
        out_specs=[
            pl.BlockSpec((RBS, D), lambda i: (i, 0)),
            pl.BlockSpec((RBS, 1), lambda i: (i, 0)),
        ],
        out_shape=[
            jax.ShapeDtypeStruct((N, D), _F32),
            jax.ShapeDtypeStruct((N, 1), _F32),
        ],
    )(x, hist, W_proj, b_proj, W_g0)


def _tcmid_call(s2, hp, dinv, b_prev, W_next):
    def body(s_ref, hp_ref, dinv_ref, b_ref, w_ref, out_ref):
        dinv = dinv_ref[...]
        h = jnp.maximum(
            dinv * (s_ref[0] + s_ref[1] + hp_ref[...]) + b_ref[...], 0.0)
        out_ref[...] = jnp.dot(dinv * h, w_ref[...],
                               preferred_element_type=_F32)

    return pl.pallas_call(
        body,
        grid=(RB,),
        in_specs=[
            pl.BlockSpec((NC, RBS, D), lambda i: (0, i, 0)),
            pl.BlockSpec((RBS, D), lambda i: (i, 0)),
            pl.BlockSpec((RBS, 1), lambda i: (i, 0)),
            pl.BlockSpec((1, D), lambda i: (0, 0)),
            pl.BlockSpec((D, D), lambda i: (0, 0)),
        ],
        out_specs=pl.BlockSpec((RBS, D), lambda i: (i, 0)),
        out_shape=jax.ShapeDtypeStruct((N, D), _F32),
    )(s2, hp, dinv, b_prev, W_next)


def _tcf_call(s2, hp, dinv, b_prev, batch, W_c1, b_c1, W_c2, b_c2):
    def body(s_ref, hp_ref, dinv_ref, b_ref, batch_ref, wc1_ref, bc1_ref,
             wc2_ref, bc2_ref, out_ref, acc, cnt):
        i = pl.program_id(0)

        @pl.when(i == 0)
        def _():
            acc[...] = jnp.zeros((NB, D), _F32)
            cnt[...] = jnp.zeros((NB, 1), _F32)

        dinv = dinv_ref[...]
        h = jnp.maximum(
            dinv * (s_ref[0] + s_ref[1] + hp_ref[...]) + b_ref[...], 0.0)
        valid_col = (i * RBS
                     + lax.broadcasted_iota(jnp.int32, (RBS, 1), 0)) < N
        h = jnp.where(valid_col, h, 0.0)
        valid_row = (i * RBS
                     + lax.broadcasted_iota(jnp.int32, (1, RBS), 1)) < N
        bb = batch_ref[pl.ds(i * RBS, RBS)]
        mask = jnp.where(
            (lax.broadcasted_iota(jnp.int32, (NB, RBS), 0) == bb[None, :])
            & valid_row, 1.0, 0.0)
        acc[...] += jnp.dot(mask, h, preferred_element_type=_F32)
        cnt[...] += jnp.sum(mask, axis=1, keepdims=True)

        @pl.when(i == RB - 1)
        def _():
            pooled = acc[...] / jnp.maximum(cnt[...], 1.0)
            z = jnp.maximum(
                jnp.dot(pooled, wc1_ref[...], preferred_element_type=_F32)
                + bc1_ref[...], 0.0)
            out_ref[...] = jnp.dot(z, wc2_ref[...],
                                   preferred_element_type=_F32) + bc2_ref[...]

    return pl.pallas_call(
        body,
        grid=(RB,),
        in_specs=[
            pl.BlockSpec((NC, RBS, D), lambda i: (0, i, 0)),
            pl.BlockSpec((RBS, D), lambda i: (i, 0)),
            pl.BlockSpec((RBS, 1), lambda i: (i, 0)),
            pl.BlockSpec((1, D), lambda i: (0, 0)),
            pl.BlockSpec((NP,), lambda i: (0,)),
            pl.BlockSpec((D, D // 2), lambda i: (0, 0)),
            pl.BlockSpec((1, D // 2), lambda i: (0, 0)),
            pl.BlockSpec((D // 2, 2), lambda i: (0, 0)),
            pl.BlockSpec((1, 2), lambda i: (0, 0)),
        ],
        out_specs=pl.BlockSpec((NB, 2), lambda i: (0, 0)),
        out_shape=jax.ShapeDtypeStruct((NB, 2), _F32),
        scratch_shapes=[
            pltpu.VMEM((NB, D), _F32),
            pltpu.VMEM((NB, 1), _F32),
        ],
    )(s2, hp, dinv, b_prev, batch, W_c1, b_c1, W_c2, b_c2)


def kernel(x, edge_index, batch, W_proj, b_proj, W_g0, b_g0, W_g1, b_g1,
           W_g2, b_g2, W_c1, b_c1, W_c2, b_c2):
    src = edge_index[0]
    dst3 = edge_index[1].reshape(NW, NCH, K)
    hist = _hist_call(dst3).reshape(NC, NP, 1)
    batch_p = jnp.pad(batch, (0, NP - N))
    hp0, dinv = _tc0_call(x, hist, W_proj, b_proj.reshape(1, D), W_g0)
    s0 = _agg_call(hp0, src, dst3)
    hp1 = _tcmid_call(s0, hp0, dinv, b_g0.reshape(1, D), W_g1)
    s1 = _agg_call(hp1, src, dst3)
    hp2 = _tcmid_call(s1, hp1, dinv, b_g1.reshape(1, D), W_g2)
    s2 = _agg_call(hp2, src, dst3)
    return _tcf_call(s2, hp2, dinv, b_g2.reshape(1, D), batch_p,
                     W_c1, b_c1.reshape(1, D // 2), W_c2, b_c2.reshape(1, 2))


# speedup vs baseline: 32.9307x; 1.2231x over previous
import functools

import jax
import jax.numpy as jnp
from jax import lax
from jax.experimental import pallas as pl
from jax.experimental.pallas import tpu as pltpu
from jax.experimental.pallas import tpu_sc as plsc

N = 10000
E = 320000
D = 128
NB = 16
NC = 2
NS = 16
NW = NC * NS
EW = E // NW
K = 80
NCH = EW // K
RB = 10
RBS = 1024
NP = 10240
RPT = NP // NS

_F32 = jnp.float32


def _sc_mesh():
    return plsc.VectorSubcoreMesh(core_axis_name="c", subcore_axis_name="s")


def _hist_call(dst3):
    @functools.partial(
        pl.kernel,
        out_type=jax.ShapeDtypeStruct((NC, NP), _F32),
        mesh=_sc_mesh(),
        scratch_types=[
            pltpu.VMEM_SHARED((NP,), _F32),
            pltpu.VMEM((640,), _F32),
            pltpu.VMEM((K,), _F32),
            pltpu.VMEM((NCH, K), jnp.int32),
            pltpu.SemaphoreType.DMA,
        ],
    )
    def hist_kernel(dst_hbm, out_hbm, hist_sh, zbuf, ones_v, didx, sem):
        c = lax.axis_index("c")
        s = lax.axis_index("s")
        w = c * NS + s

        def fill(i, _):
            zbuf[pl.ds(i * 16, 16)] = jnp.zeros((16,), _F32)
            return 0

        lax.fori_loop(0, 40, fill, 0)
        for i in range(K // 16):
            ones_v[pl.ds(i * 16, 16)] = jnp.ones((16,), _F32)

        pltpu.sync_copy(zbuf, hist_sh.at[pl.ds(s * 640, 640)])
        pltpu.sync_copy(dst_hbm.at[w], didx)
        plsc.subcore_barrier()

        def fire(it, _):
            pltpu.async_copy(ones_v, hist_sh.at[didx.at[it]], sem, add=True)
            return 0

        lax.fori_loop(0, NCH, fire, 0)

        def drain(it, _):
            pltpu.make_async_copy(ones_v, hist_sh.at[didx.at[it]], sem).wait()
            return 0

        lax.fori_loop(0, NCH, drain, 0)
        plsc.subcore_barrier()
        pltpu.sync_copy(hist_sh.at[pl.ds(s * 640, 640)],
                        out_hbm.at[c, pl.ds(s * 640, 640)])

    return hist_kernel(dst3)


NBUF = 4
NFULL = NCH // NBUF
NREM = NCH % NBUF


def _agg_call(hp, src, dst4):
    @functools.partial(
        pl.kernel,
        out_type=jax.ShapeDtypeStruct((NC, NP, D), _F32),
        mesh=_sc_mesh(),
        scratch_types=[
            pltpu.VMEM_SHARED((NP, D), _F32),
            pltpu.VMEM((8, D), _F32),
        ] + [pltpu.VMEM((K,), jnp.int32)] * NBUF
          + [pltpu.VMEM((1, K), jnp.int32)] * NBUF
          + [pltpu.VMEM((K, D), _F32)] * NBUF
          + [pltpu.SemaphoreType.DMA] * (4 * NBUF),
    )
    def agg_kernel(hp_hbm, src_hbm, dst_hbm, out_hbm, acc, zbuf,
                   sb0, sb1, sb2, sb3, db0, db1, db2, db3,
                   r0, r1, r2, r3,
                   gs0, gs1, gs2, gs3, ts0, ts1, ts2, ts3,
                   us0, us1, us2, us3, vs0, vs1, vs2, vs3):
        sbuf = [sb0, sb1, sb2, sb3]
        dbuf = [db0, db1, db2, db3]
        rows = [r0, r1, r2, r3]
        gsem = [gs0, gs1, gs2, gs3]
        ssem = [ts0, ts1, ts2, ts3]
        issem = [us0, us1, us2, us3]
        idsem = [vs0, vs1, vs2, vs3]
        c = lax.axis_index("c")
        s = lax.axis_index("s")
        w = c * NS + s

        def il_s(j, b):
            pltpu.async_copy(src_hbm.at[pl.ds(w * EW + j * K, K)], sbuf[b],
                             issem[b])

        def il_d(j, b):
            pltpu.async_copy(dst_hbm.at[w, pl.ds(j, 1)], dbuf[b], idsem[b])

        def wait_is(b):
            pltpu.make_async_copy(src_hbm.at[pl.ds(w * EW, K)], sbuf[b],
                                  issem[b]).wait()

        def wait_id(b):
            pltpu.make_async_copy(dst_hbm.at[w, pl.ds(0, 1)], dbuf[b],
                                  idsem[b]).wait()

        def fire_g(b):
            pltpu.async_copy(hp_hbm.at[sbuf[b]], rows[b], gsem[b])

        for b in range(NBUF):
            il_s(b, b)
        for b in range(NBUF):
            il_d(b, b)
        for b in range(NBUF):
            wait_is(b)
            fire_g(b)

        def fill(i, _):
            zbuf[i // 8, pl.ds((i % 8) * 16, 16)] = jnp.zeros((16,), _F32)
            return 0

        lax.fori_loop(0, 64, fill, 0)

        def zero(j, _):
            pltpu.sync_copy(zbuf, acc.at[pl.ds(s * RPT + j * 8, 8)])
            return 0

        lax.fori_loop(0, RPT // 8, zero, 0)
        plsc.subcore_barrier()

        def step(j, b, pf):
            pltpu.make_async_copy(hp_hbm.at[sbuf[b]], rows[b],
                                  gsem[b]).wait()
            if pf:
                il_s(j + NBUF, b)
            wait_id(b)
            pltpu.async_copy(rows[b], acc.at[dbuf[b].at[0]], ssem[b],
                             add=True)
            pltpu.make_async_copy(rows[b], acc.at[dbuf[b].at[0]],
                                  ssem[b]).wait()
            if pf:
                il_d(j + NBUF, b)
                wait_is(b)
                fire_g(b)

        def outer(g, _):
            for b in range(NBUF):
                step(g * NBUF + b, b, True)
            return 0

        lax.fori_loop(0, NFULL - 1, outer, 0)
        for b in range(NBUF):
            step((NFULL - 1) * NBUF + b, b, b < NREM)
        for b in range(NREM):
            step(NFULL * NBUF + b, b, False)

        plsc.subcore_barrier()
        pltpu.sync_copy(acc.at[pl.ds(s * RPT, RPT)],
                        out_hbm.at[c, pl.ds(s * RPT, RPT)])

    return agg_kernel(hp, src, dst4)


def _tc0_call(x, hist, W_proj, b_proj, W_g0):
    def body(x_ref, hist_ref, wp_ref, bp_ref, wg_ref, hp_ref, dinv_ref):
        dinv = lax.rsqrt(1.0 + hist_ref[0] + hist_ref[1])
        h = jnp.maximum(
            jnp.dot(x_ref[...], wp_ref[...], preferred_element_type=_F32)
            + bp_ref[...], 0.0)
        hp_ref[...] = jnp.dot(dinv * h, wg_ref[...],
                              preferred_element_type=_F32)
        dinv_ref[...] = dinv

    return pl.pallas_call(
        body,
        grid=(RB,),
        in_specs=[
            pl.BlockSpec((RBS, D), lambda i: (i, 0)),
            pl.BlockSpec((NC, RBS, 1), lambda i: (0, i, 0)),
            pl.BlockSpec((D, D), lambda i: (0, 0)),
            pl.BlockSpec((1, D), lambda i: (0, 0)),
            pl.BlockSpec((D, D), lambda i: (0, 0)),
        ],
        out_specs=[
            pl.BlockSpec((RBS, D), lambda i: (i, 0)),
            pl.BlockSpec((RBS, 1), lambda i: (i, 0)),
        ],
        out_shape=[
            jax.ShapeDtypeStruct((N, D), _F32),
            jax.ShapeDtypeStruct((N, 1), _F32),
        ],
    )(x, hist, W_proj, b_proj, W_g0)


def _tcmid_call(s2, hp, dinv, b_prev, W_next):
    def body(s_ref, hp_ref, dinv_ref, b_ref, w_ref, out_ref):
        dinv = dinv_ref[...]
        h = jnp.maximum(
            dinv * (s_ref[0] + s_ref[1] + hp_ref[...]) + b_ref[...], 0.0)
        out_ref[...] = jnp.dot(dinv * h, w_ref[...],
                               preferred_element_type=_F32)

    return pl.pallas_call(
        body,
        grid=(RB,),
        in_specs=[
            pl.BlockSpec((NC, RBS, D), lambda i: (0, i, 0)),
            pl.BlockSpec((RBS, D), lambda i: (i, 0)),
            pl.BlockSpec((RBS, 1), lambda i: (i, 0)),
            pl.BlockSpec((1, D), lambda i: (0, 0)),
            pl.BlockSpec((D, D), lambda i: (0, 0)),
        ],
        out_specs=pl.BlockSpec((RBS, D), lambda i: (i, 0)),
        out_shape=jax.ShapeDtypeStruct((N, D), _F32),
    )(s2, hp, dinv, b_prev, W_next)


def _tcf_call(s2, hp, dinv, b_prev, batch, W_c1, b_c1, W_c2, b_c2):
    def body(s_ref, hp_ref, dinv_ref, b_ref, batch_ref, wc1_ref, bc1_ref,
             wc2_ref, bc2_ref, out_ref, acc, cnt):
        i = pl.program_id(0)

        @pl.when(i == 0)
        def _():
            acc[...] = jnp.zeros((NB, D), _F32)
            cnt[...] = jnp.zeros((NB, 1), _F32)

        dinv = dinv_ref[...]
        h = jnp.maximum(
            dinv * (s_ref[0] + s_ref[1] + hp_ref[...]) + b_ref[...], 0.0)
        valid_col = (i * RBS
                     + lax.broadcasted_iota(jnp.int32, (RBS, 1), 0)) < N
        h = jnp.where(valid_col, h, 0.0)
        valid_row = (i * RBS
                     + lax.broadcasted_iota(jnp.int32, (1, RBS), 1)) < N
        bb = batch_ref[pl.ds(i * RBS, RBS)]
        mask = jnp.where(
            (lax.broadcasted_iota(jnp.int32, (NB, RBS), 0) == bb[None, :])
            & valid_row, 1.0, 0.0)
        acc[...] += jnp.dot(mask, h, preferred_element_type=_F32)
        cnt[...] += jnp.sum(mask, axis=1, keepdims=True)

        @pl.when(i == RB - 1)
        def _():
            pooled = acc[...] / jnp.maximum(cnt[...], 1.0)
            z = jnp.maximum(
                jnp.dot(pooled, wc1_ref[...], preferred_element_type=_F32)
                + bc1_ref[...], 0.0)
            out_ref[...] = jnp.dot(z, wc2_ref[...],
                                   preferred_element_type=_F32) + bc2_ref[...]

    return pl.pallas_call(
        body,
        grid=(RB,),
        in_specs=[
            pl.BlockSpec((NC, RBS, D), lambda i: (0, i, 0)),
            pl.BlockSpec((RBS, D), lambda i: (i, 0)),
            pl.BlockSpec((RBS, 1), lambda i: (i, 0)),
            pl.BlockSpec((1, D), lambda i: (0, 0)),
            pl.BlockSpec((NP,), lambda i: (0,)),
            pl.BlockSpec((D, D // 2), lambda i: (0, 0)),
            pl.BlockSpec((1, D // 2), lambda i: (0, 0)),
            pl.BlockSpec((D // 2, 2), lambda i: (0, 0)),
            pl.BlockSpec((1, 2), lambda i: (0, 0)),
        ],
        out_specs=pl.BlockSpec((NB, 2), lambda i: (0, 0)),
        out_shape=jax.ShapeDtypeStruct((NB, 2), _F32),
        scratch_shapes=[
            pltpu.VMEM((NB, D), _F32),
            pltpu.VMEM((NB, 1), _F32),
        ],
    )(s2, hp, dinv, b_prev, batch, W_c1, b_c1, W_c2, b_c2)


def kernel(x, edge_index, batch, W_proj, b_proj, W_g0, b_g0, W_g1, b_g1,
           W_g2, b_g2, W_c1, b_c1, W_c2, b_c2):
    src = edge_index[0]
    dst3 = edge_index[1].reshape(NW, NCH, K)
    hist = _hist_call(dst3).reshape(NC, NP, 1)
    batch_p = jnp.pad(batch, (0, NP - N))
    hp0, dinv = _tc0_call(x, hist, W_proj, b_proj.reshape(1, D), W_g0)
    s0 = _agg_call(hp0, src, dst3)
    hp1 = _tcmid_call(s0, hp0, dinv, b_g0.reshape(1, D), W_g1)
    s1 = _agg_call(hp1, src, dst3)
    hp2 = _tcmid_call(s1, hp1, dinv, b_g1.reshape(1, D), W_g2)
    s2 = _agg_call(hp2, src, dst3)
    return _tcf_call(s2, hp2, dinv, b_g2.reshape(1, D), batch_p,
                     W_c1, b_c1.reshape(1, D // 2), W_c2, b_c2.reshape(1, 2))
